# baseline (speedup 1.0000x reference)
import functools

import jax
import jax.numpy as jnp
from jax.experimental import pallas as pl
from jax.experimental.pallas import tpu as pltpu

D_IN = 768
D_OUT = 768
LORA_R = 64
N_EXP = 64
TILE_T = 512
ER = N_EXP * LORA_R


def _fused_body(x_ref, wb_ref, bb_ref, a_ref, rw_ref, b2_ref, o_ref,
                se_ref, wbf_ref):
    pid = pl.program_id(0)

    @pl.when(pid == 0)
    def _prep():
        col = jax.lax.broadcasted_iota(jnp.int32, (N_EXP, ER), 1)
        row = jax.lax.broadcasted_iota(jnp.int32, (N_EXP, ER), 0)
        se_ref[...] = (jax.lax.shift_right_logical(col, 6) == row
                       ).astype(jnp.bfloat16)
        wbf_ref[...] = wb_ref[...].astype(jnp.bfloat16)

    xb = x_ref[0].astype(jnp.bfloat16)

    logits = jax.lax.dot_general(
        xb, rw_ref[...].astype(jnp.bfloat16), (((1,), (1,)), ((), ())),
        preferred_element_type=jnp.float32)
    m = jnp.max(logits, axis=1, keepdims=True)
    p = jnp.exp(logits - m)
    p = p / jnp.sum(p, axis=1, keepdims=True)

    lane = jax.lax.broadcasted_iota(jnp.int32, p.shape, 1)
    m1 = jnp.max(p, axis=1, keepdims=True)
    i1 = jnp.min(jnp.where(p == m1, lane, N_EXP), axis=1, keepdims=True)
    p_ex = jnp.where(lane == i1, -1.0, p)
    m2 = jnp.max(p_ex, axis=1, keepdims=True)
    i2 = jnp.min(jnp.where(p_ex == m2, lane, N_EXP), axis=1, keepdims=True)
    keep = (lane == i1) | (lane == i2)
    wv = jnp.where(keep, p, 0.0) / (m1 + m2 + 1e-6)

    h = jax.lax.dot_general(
        xb, a_ref[...].astype(jnp.bfloat16), (((1,), (1,)), ((), ())),
        preferred_element_type=jnp.float32)

    hb16 = h.astype(jnp.bfloat16)
    h_rep = jnp.concatenate([hb16] * N_EXP, axis=1)
    w_rep = jax.lax.dot_general(
        wv.astype(jnp.bfloat16), se_ref[...], (((1,), (0,)), ((), ())),
        preferred_element_type=jnp.float32)
    pmat = h_rep * w_rep.astype(jnp.bfloat16)

    delta = jax.lax.dot_general(
        pmat, b2_ref[...], (((1,), (1,)), ((), ())),
        preferred_element_type=jnp.float32)

    base = jax.lax.dot_general(
        xb, wbf_ref[...], (((1,), (1,)), ((), ())),
        preferred_element_type=jnp.float32)
    o_ref[0] = base + bb_ref[...] + delta


@functools.partial(jax.jit, static_argnames=("interpret",))
def kernel(x, W_base, b_base, A, B, router_w, interpret=False):
    t = x.shape[1]
    b2 = jnp.transpose(B.astype(jnp.bfloat16), (1, 0, 2)).reshape(D_OUT, ER)
    out = pl.pallas_call(
        _fused_body,
        grid=(t // TILE_T,),
        in_specs=[
            pl.BlockSpec((1, TILE_T, D_IN), lambda i: (0, i, 0)),
            pl.BlockSpec((D_OUT, D_IN), lambda i: (0, 0)),
            pl.BlockSpec((1, D_OUT), lambda i: (0, 0)),
            pl.BlockSpec((LORA_R, D_IN), lambda i: (0, 0)),
            pl.BlockSpec((N_EXP, D_IN), lambda i: (0, 0)),
            pl.BlockSpec((D_OUT, ER), lambda i: (0, 0)),
        ],
        out_specs=pl.BlockSpec((1, TILE_T, D_OUT), lambda i: (0, i, 0)),
        out_shape=jax.ShapeDtypeStruct((1, t, D_OUT), jnp.float32),
        scratch_shapes=[
            pltpu.VMEM((N_EXP, ER), jnp.bfloat16),
            pltpu.VMEM((D_OUT, D_IN), jnp.bfloat16),
        ],
        interpret=interpret,
    )(x, W_base, b_base.reshape(1, D_OUT), A, router_w, b2)
    return out


# speedup vs baseline: 1.2525x; 1.2525x over previous
import functools

import jax
import jax.numpy as jnp
from jax.experimental import pallas as pl
from jax.experimental.pallas import tpu as pltpu

D_IN = 768
D_OUT = 768
LORA_R = 64
N_EXP = 64
TILE_T = 1024
ER = N_EXP * LORA_R


def _fused_body(x_ref, wb_ref, bb_ref, a_ref, rw_ref, b2_ref, o_ref,
                se_ref, wbf_ref):
    pid = pl.program_id(0)

    @pl.when(pid == 0)
    def _prep():
        col = jax.lax.broadcasted_iota(jnp.int32, (N_EXP, ER), 1)
        row = jax.lax.broadcasted_iota(jnp.int32, (N_EXP, ER), 0)
        se_ref[...] = (jax.lax.shift_right_logical(col, 6) == row
                       ).astype(jnp.bfloat16)
        wbf_ref[...] = wb_ref[...].astype(jnp.bfloat16)

    xb = x_ref[0].astype(jnp.bfloat16)

    logits = jax.lax.dot_general(
        xb, rw_ref[...].astype(jnp.bfloat16), (((1,), (1,)), ((), ())),
        preferred_element_type=jnp.float32)
    m = jnp.max(logits, axis=1, keepdims=True)
    p = jnp.exp(logits - m)
    p = p / jnp.sum(p, axis=1, keepdims=True)

    lane = jax.lax.broadcasted_iota(jnp.int32, p.shape, 1)
    m1 = jnp.max(p, axis=1, keepdims=True)
    i1 = jnp.min(jnp.where(p == m1, lane, N_EXP), axis=1, keepdims=True)
    p_ex = jnp.where(lane == i1, -1.0, p)
    m2 = jnp.max(p_ex, axis=1, keepdims=True)
    i2 = jnp.min(jnp.where(p_ex == m2, lane, N_EXP), axis=1, keepdims=True)
    keep = (lane == i1) | (lane == i2)
    wv = jnp.where(keep, p, 0.0) / (m1 + m2 + 1e-6)

    h = jax.lax.dot_general(
        xb, a_ref[...].astype(jnp.bfloat16), (((1,), (1,)), ((), ())),
        preferred_element_type=jnp.float32)

    hb16 = h.astype(jnp.bfloat16)
    h_rep = jnp.concatenate([hb16] * N_EXP, axis=1)
    w_rep = jax.lax.dot_general(
        wv.astype(jnp.bfloat16), se_ref[...], (((1,), (0,)), ((), ())),
        preferred_element_type=jnp.float32)
    pmat = h_rep * w_rep.astype(jnp.bfloat16)

    delta = jax.lax.dot_general(
        pmat, b2_ref[...], (((1,), (0,)), ((), ())),
        preferred_element_type=jnp.float32)

    base = jax.lax.dot_general(
        xb, wbf_ref[...], (((1,), (1,)), ((), ())),
        preferred_element_type=jnp.float32)
    o_ref[0] = base + bb_ref[...] + delta


@functools.partial(jax.jit, static_argnames=("interpret",))
def kernel(x, W_base, b_base, A, B, router_w, interpret=False):
    t = x.shape[1]
    b2 = jnp.transpose(B.astype(jnp.bfloat16), (0, 2, 1)).reshape(ER, D_OUT)
    out = pl.pallas_call(
        _fused_body,
        grid=(t // TILE_T,),
        in_specs=[
            pl.BlockSpec((1, TILE_T, D_IN), lambda i: (0, i, 0)),
            pl.BlockSpec((D_OUT, D_IN), lambda i: (0, 0)),
            pl.BlockSpec((1, D_OUT), lambda i: (0, 0)),
            pl.BlockSpec((LORA_R, D_IN), lambda i: (0, 0)),
            pl.BlockSpec((N_EXP, D_IN), lambda i: (0, 0)),
            pl.BlockSpec((ER, D_OUT), lambda i: (0, 0)),
        ],
        out_specs=pl.BlockSpec((1, TILE_T, D_OUT), lambda i: (0, i, 0)),
        out_shape=jax.ShapeDtypeStruct((1, t, D_OUT), jnp.float32),
        scratch_shapes=[
            pltpu.VMEM((N_EXP, ER), jnp.bfloat16),
            pltpu.VMEM((D_OUT, D_IN), jnp.bfloat16),
        ],
        interpret=interpret,
    )(x, W_base, b_base.reshape(1, D_OUT), A, router_w, b2)
    return out
